# Initial kernel scaffold; baseline (speedup 1.0000x reference)
import jax
import jax.numpy as jnp
from jax.experimental import pallas as pl


def kernel(Q, K, V):
    raise NotImplementedError("write your pallas kernel here")


# speedup vs baseline: 32.1690x; 32.1690x over previous
import math

import jax
import jax.numpy as jnp
from jax.experimental import pallas as pl
from jax.experimental.pallas import tpu as pltpu

K_BR = 8
N0, N1, N2, N3 = 512, 64, 8, 1
O1, O2, O3 = 512, 576, 584
NT = 585
D = 128
SCALE = 1.0 / math.sqrt(D)
NEG = -1e30


def _nt(a, b):
    return jax.lax.dot_general(a, b, (((1,), (1,)), ((), ())),
                               preferred_element_type=jnp.float32)


def _nn(a, b):
    return jax.lax.dot_general(a, b, (((1,), (0,)), ((), ())),
                               preferred_element_type=jnp.float32)


def _masked_attn(s, mask, vals, s0, cnt, v_sink):
    sm = jnp.where(mask, s, NEG)
    m = jnp.maximum(jnp.max(sm, axis=1, keepdims=True), s0)
    w = jnp.where(mask, jnp.exp(sm - m), 0.0)
    e0 = cnt * jnp.exp(s0 - m)
    num = _nn(w, vals) + e0 * v_sink
    den = jnp.sum(w, axis=1, keepdims=True) + e0
    return num / den


def _body(q_ref, k_ref, v_ref, o_ref):
    Qb = q_ref[0]
    Kb = k_ref[0]
    Vb = v_ref[0]

    k0 = Kb[0:1, :]

    q = Qb[0:N0] * SCALE
    s0 = _nt(q, k0)
    v_sink0 = Vb[0:1, :]

    ii = jax.lax.broadcasted_iota(jnp.int32, (N0, N1), 0)
    jj = jax.lax.broadcasted_iota(jnp.int32, (N0, N1), 1)
    s_anc = _nt(q, Kb[O1:O2])
    mask_anc = jj < (ii // K_BR)
    cnt_anc = (N1 - ii[:, 0:1] // K_BR).astype(jnp.float32)
    attn_anc = _masked_attn(s_anc, mask_anc, Vb[O1:O2], s0, cnt_anc, v_sink0)

    i2 = jax.lax.broadcasted_iota(jnp.int32, (N0, N0), 0)
    j2 = jax.lax.broadcasted_iota(jnp.int32, (N0, N0), 1)
    s_sib = _nt(q, Kb[0:N0])
    mask_sib = (j2 // K_BR == i2 // K_BR) & (j2 <= i2)
    cnt_sib = (K_BR - 1 - ii[:, 0:1] % K_BR).astype(jnp.float32)
    attn_sib = _masked_attn(s_sib, mask_sib, Vb[0:N0], s0, cnt_sib, v_sink0)

    out0 = (attn_anc + attn_sib) / 3.0
    v_sink = out0[0:1, :]

    q1 = Qb[O1:O2] * SCALE
    s0 = _nt(q1, k0)

    ii = jax.lax.broadcasted_iota(jnp.int32, (N1, K_BR), 0)
    jj = jax.lax.broadcasted_iota(jnp.int32, (N1, K_BR), 1)
    s_anc = _nt(q1, Kb[O2:O3])
    mask_anc = jj < (ii // K_BR)
    cnt_anc = (N2 - ii[:, 0:1] // K_BR).astype(jnp.float32)
    attn_anc = _masked_attn(s_anc, mask_anc, Vb[O2:O3], s0, cnt_anc, v_sink)

    i2 = jax.lax.broadcasted_iota(jnp.int32, (N1, N1), 0)
    j2 = jax.lax.broadcasted_iota(jnp.int32, (N1, N1), 1)
    s_sib = _nt(q1, Kb[O1:O2])
    mask_sib = (j2 // K_BR == i2 // K_BR) & (j2 <= i2)
    cnt_sib = (K_BR - 1 - ii[:, 0:1] % K_BR).astype(jnp.float32)
    attn_sib = _masked_attn(s_sib, mask_sib, Vb[O1:O2], s0, cnt_sib, v_sink)

    ic = jax.lax.broadcasted_iota(jnp.int32, (N1, N0), 0)
    jc = jax.lax.broadcasted_iota(jnp.int32, (N1, N0), 1)
    onehot = (jc == ic * K_BR).astype(jnp.float32)
    s_ch_full = _nt(q1, Kb[0:N0])
    c = jnp.sum(jnp.where(onehot > 0, s_ch_full, 0.0), axis=1, keepdims=True)
    ch_vals = _nn(onehot, out0)
    m = jnp.maximum(c, s0)
    ec = jnp.exp(c - m)
    e0 = (K_BR - 1) * jnp.exp(s0 - m)
    attn_ch = (ec * ch_vals + e0 * v_sink) / (ec + e0)

    out1 = (attn_anc + attn_sib + attn_ch) / 3.0

    q2 = Qb[O2:O3] * SCALE
    s0 = _nt(q2, k0)

    attn_anc = jnp.broadcast_to(v_sink, (N2, D))

    i2 = jax.lax.broadcasted_iota(jnp.int32, (N2, N2), 0)
    j2 = jax.lax.broadcasted_iota(jnp.int32, (N2, N2), 1)
    s_sib = _nt(q2, Kb[O2:O3])
    mask_sib = j2 <= i2
    cnt_sib = (K_BR - 1 - i2[:, 0:1]).astype(jnp.float32)
    attn_sib = _masked_attn(s_sib, mask_sib, Vb[O2:O3], s0, cnt_sib, v_sink)

    ic = jax.lax.broadcasted_iota(jnp.int32, (N2, N1), 0)
    jc = jax.lax.broadcasted_iota(jnp.int32, (N2, N1), 1)
    onehot = (jc == ic * K_BR).astype(jnp.float32)
    s_ch_full = _nt(q2, Kb[O1:O2])
    c = jnp.sum(jnp.where(onehot > 0, s_ch_full, 0.0), axis=1, keepdims=True)
    ch_vals = _nn(onehot, out1)
    m = jnp.maximum(c, s0)
    ec = jnp.exp(c - m)
    e0 = (K_BR - 1) * jnp.exp(s0 - m)
    attn_ch = (ec * ch_vals + e0 * v_sink) / (ec + e0)

    out2 = (attn_anc + attn_sib + attn_ch) / 3.0

    q3 = Qb[O3:NT] * SCALE
    s0 = _nt(q3, k0)
    attn_sib = Vb[O3:NT]
    c = _nt(q3, Kb[O2:O2 + 1])
    m = jnp.maximum(c, s0)
    ec = jnp.exp(c - m)
    e0 = (K_BR - 1) * jnp.exp(s0 - m)
    attn_ch = (ec * out2[0:1, :] + e0 * v_sink) / (ec + e0)

    out3 = (attn_sib + attn_ch) / 3.0

    o_ref[0] = jnp.concatenate([out0, out1, out2, out3], axis=0)


def kernel(Q, K, V):
    B, H, N, d = Q.shape
    BH = B * H
    Qr = Q.reshape(BH, N, d)
    Kr = K.reshape(BH, N, d)
    Vr = V.reshape(BH, N, d)
    spec = pl.BlockSpec((1, N, d), lambda i: (i, 0, 0))
    out = pl.pallas_call(
        _body,
        grid=(BH,),
        in_specs=[spec, spec, spec],
        out_specs=spec,
        out_shape=jax.ShapeDtypeStruct((BH, N, d), jnp.float32),
        compiler_params=pltpu.CompilerParams(
            dimension_semantics=("parallel",),
        ),
    )(Qr, Kr, Vr)
    return out.reshape(B, H, N, d)
